# baseline (speedup 1.0000x reference)
import functools

import jax
import jax.numpy as jnp
from jax import lax
from jax.experimental import pallas as pl
from jax.experimental.pallas import tpu as pltpu
from jax.experimental.pallas import tpu_sc as plsc

VOCAB = 1000
EMB_DIM = 128
HID = 8

_NC = 2
_NS = 16
_NW = _NC * _NS
_CH = 128
_BB = 2048


def _tables_body(emb_ref, w_ref, out_ref):
    e = emb_ref[...]
    wa = w_ref[:, :EMB_DIM]
    wb = w_ref[:, EMB_DIM:]
    dn = (((1,), (1,)), ((), ()))
    out_ref[:VOCAB, :] = lax.dot_general(e, wa, dn, preferred_element_type=jnp.float32)
    out_ref[VOCAB:, :] = lax.dot_general(e, wb, dn, preferred_element_type=jnp.float32)


def _build_tables(emb, fc1_w):
    return pl.pallas_call(
        _tables_body,
        out_shape=jax.ShapeDtypeStruct((2 * VOCAB, HID), jnp.float32),
    )(emb, fc1_w)


_BLANE = 16


def _sc_gather(table, idx2, batch):
    nchunks = 2 * batch // _CH
    per_w = nchunks // _NW
    half_c = nchunks // 2
    mesh = plsc.VectorSubcoreMesh(core_axis_name="c", subcore_axis_name="s")

    @functools.partial(
        pl.kernel,
        mesh=mesh,
        compiler_params=pltpu.CompilerParams(use_tc_tiling_on_sc=False),
        out_type=jax.ShapeDtypeStruct((batch, 128), jnp.float32),
        scratch_types=[
            pltpu.VMEM((per_w, _CH), jnp.int32),
            pltpu.VMEM((per_w * _CH, HID), jnp.float32),
            pltpu.SemaphoreType.DMA,
        ],
    )
    def k(table_hbm, idx_hbm, out_hbm, idx_v, rows_v, sem):
        wid = lax.axis_index("s") * _NC + lax.axis_index("c")
        cbase = wid * per_w
        is_a = cbase < half_c
        lane0 = jnp.where(is_a, 0, _BLANE)
        rsub = jnp.where(is_a, 0, half_c * _CH)
        pltpu.sync_copy(idx_hbm.at[pl.ds(cbase, per_w)], idx_v)
        copies = [
            pltpu.async_copy(table_hbm.at[idx_v.at[j]],
                             rows_v.at[pl.ds(j * _CH, _CH)], sem)
            for j in range(per_w)
        ]
        for c in copies:
            c.wait()
        pltpu.sync_copy(
            rows_v,
            out_hbm.at[pl.ds(cbase * _CH - rsub, per_w * _CH),
                       pl.ds(lane0, HID)])

    return k(table, idx2)


def _mlp_body(g_ref, b1_ref, w2_ref, out_ref):
    g = g_ref[...]
    t = jnp.tanh(g + pltpu.roll(g, 128 - _BLANE, 1) + b1_ref[...])
    lane = lax.broadcasted_iota(jnp.int32, (_BB, 128), 1)
    t = jnp.where(lane < HID, t, jnp.where(lane == HID, 1.0, 0.0))
    out_ref[...] = lax.dot_general(w2_ref[...], t, (((0,), (1,)), ((), ())),
                                   preferred_element_type=jnp.float32)


def _mlp(g2, fc1_b, fc2_w, fc2_b, batch):
    nb = batch // _BB
    b1p = jnp.pad(fc1_b, (0, 128 - HID)).reshape(1, 128)
    w2p = jnp.concatenate(
        [fc2_w.T, fc2_b[None, :], jnp.zeros((128 - HID - 1, VOCAB), jnp.float32)], 0)
    out_t = pl.pallas_call(
        _mlp_body,
        grid=(nb,),
        in_specs=[
            pl.BlockSpec((_BB, 128), lambda i: (i, 0)),
            pl.BlockSpec((1, 128), lambda i: (0, 0)),
            pl.BlockSpec((128, VOCAB), lambda i: (0, 0)),
        ],
        out_specs=pl.BlockSpec((VOCAB, _BB), lambda i: (0, i)),
        out_shape=jax.ShapeDtypeStruct((VOCAB, batch), jnp.float32),
    )(g2, b1p, w2p)
    return out_t.T


def kernel(x, emb, fc1_w, fc1_b, fc2_w, fc2_b):
    batch = x.shape[0]
    table = _build_tables(emb, fc1_w)
    xi = x.astype(jnp.int32)
    idx = (xi.T + jnp.array([[0], [VOCAB]], jnp.int32)).reshape(-1, _CH)
    g = _sc_gather(table, idx, batch)
    return _mlp(g, fc1_b, fc2_w, fc2_b, batch)


# speedup vs baseline: 1.0005x; 1.0005x over previous
import functools

import jax
import jax.numpy as jnp
from jax import lax
from jax.experimental import pallas as pl
from jax.experimental.pallas import tpu as pltpu
from jax.experimental.pallas import tpu_sc as plsc

VOCAB = 1000
EMB_DIM = 128
HID = 8

_NC = 2
_NS = 16
_NW = _NC * _NS
_CH = 128
_BB = 2048


def _tables_body(emb_ref, w_ref, out_ref):
    e = emb_ref[...]
    wa = w_ref[:, :EMB_DIM]
    wb = w_ref[:, EMB_DIM:]
    dn = (((1,), (1,)), ((), ()))
    out_ref[:VOCAB, :] = lax.dot_general(e, wa, dn, preferred_element_type=jnp.float32)
    out_ref[VOCAB:, :] = lax.dot_general(e, wb, dn, preferred_element_type=jnp.float32)


def _build_tables(emb, fc1_w):
    return pl.pallas_call(
        _tables_body,
        out_shape=jax.ShapeDtypeStruct((2 * VOCAB, HID), jnp.float32),
    )(emb, fc1_w)


_BLANE = 16


def _sc_gather(table, idx2, batch):
    nchunks = 2 * batch // _CH
    per_w = nchunks // _NW
    half_c = nchunks // 2
    mesh = plsc.VectorSubcoreMesh(core_axis_name="c", subcore_axis_name="s")

    @functools.partial(
        pl.kernel,
        mesh=mesh,
        compiler_params=pltpu.CompilerParams(use_tc_tiling_on_sc=False),
        out_type=jax.ShapeDtypeStruct((batch, 128), jnp.float32),
        scratch_types=[
            pltpu.VMEM((per_w, _CH), jnp.int32),
            pltpu.VMEM((per_w * _CH, HID), jnp.float32),
            pltpu.SemaphoreType.DMA,
        ],
    )
    def k(table_hbm, idx_hbm, out_hbm, idx_v, rows_v, sem):
        wid = lax.axis_index("s") * _NC + lax.axis_index("c")
        cbase = wid * per_w
        is_a = cbase < half_c
        lane0 = jnp.where(is_a, 0, _BLANE)
        rsub = jnp.where(is_a, 0, half_c * _CH)
        pltpu.sync_copy(idx_hbm.at[pl.ds(cbase, per_w)], idx_v)
        copies = [
            pltpu.async_copy(table_hbm.at[idx_v.at[j]],
                             rows_v.at[pl.ds(j * _CH, _CH)], sem)
            for j in range(per_w)
        ]
        for c in copies:
            c.wait()
        pltpu.sync_copy(
            rows_v,
            out_hbm.at[pl.ds(cbase * _CH - rsub, per_w * _CH),
                       pl.ds(lane0, HID)])

    return k(table, idx2)


def _mlp_body(g_ref, b1_ref, w2_ref, out_ref):
    g = g_ref[...]
    t = jnp.tanh(g + pltpu.roll(g, 128 - _BLANE, 1) + b1_ref[...])
    lane = lax.broadcasted_iota(jnp.int32, (_BB, 128), 1)
    t = jnp.where(lane < HID, t, jnp.where(lane == HID, 1.0, 0.0))
    out_ref[...] = lax.dot_general(w2_ref[...], t, (((0,), (1,)), ((), ())),
                                   preferred_element_type=jnp.float32)


def _mlp(g2, fc1_b, fc2_w, fc2_b, batch):
    nb = batch // _BB
    b1p = jnp.pad(fc1_b, (0, 128 - HID)).reshape(1, 128)
    w2p = jnp.concatenate(
        [fc2_w.T, fc2_b[None, :], jnp.zeros((128 - HID - 1, VOCAB), jnp.float32)], 0)
    out_t = pl.pallas_call(
        _mlp_body,
        grid=(nb,),
        in_specs=[
            pl.BlockSpec((_BB, 128), lambda i: (i, 0)),
            pl.BlockSpec((1, 128), lambda i: (0, 0)),
            pl.BlockSpec((128, VOCAB), lambda i: (0, 0)),
        ],
        out_specs=pl.BlockSpec((VOCAB, _BB), lambda i: (0, i)),
        out_shape=jax.ShapeDtypeStruct((VOCAB, batch), jnp.float32),
    )(g2, b1p, w2p)
    return out_t.T


def kernel(x, emb, fc1_w, fc1_b, fc2_w, fc2_b):
    batch = x.shape[0]
    table = _build_tables(emb, fc1_w)
    xi = x.astype(jnp.int32)
    idx = jnp.concatenate([xi[:, 0], xi[:, 1] + VOCAB]).reshape(-1, _CH)
    g = _sc_gather(table, idx, batch)
    return _mlp(g, fc1_b, fc2_w, fc2_b, batch)


# speedup vs baseline: 1.0114x; 1.0108x over previous
import functools

import jax
import jax.numpy as jnp
from jax import lax
from jax.experimental import pallas as pl
from jax.experimental.pallas import tpu as pltpu
from jax.experimental.pallas import tpu_sc as plsc

VOCAB = 1000
EMB_DIM = 128
HID = 8

_NC = 2
_NS = 16
_NW = _NC * _NS
_CH = 128
_BB = 2048


def _tables_body(emb_ref, w_ref, out_ref):
    e = emb_ref[...]
    wa = w_ref[:, :EMB_DIM]
    wb = w_ref[:, EMB_DIM:]
    dn = (((1,), (1,)), ((), ()))
    out_ref[:VOCAB, :] = lax.dot_general(e, wa, dn, preferred_element_type=jnp.float32)
    out_ref[VOCAB:, :] = lax.dot_general(e, wb, dn, preferred_element_type=jnp.float32)


def _build_tables(emb, fc1_w):
    return pl.pallas_call(
        _tables_body,
        out_shape=jax.ShapeDtypeStruct((2 * VOCAB, HID), jnp.float32),
    )(emb, fc1_w)


_BLANE = 16


def _sc_gather(table, idx2, batch):
    nchunks = 2 * batch // _CH
    per_w = nchunks // _NW
    half_c = nchunks // 2
    mesh = plsc.VectorSubcoreMesh(core_axis_name="c", subcore_axis_name="s")

    @functools.partial(
        pl.kernel,
        mesh=mesh,
        compiler_params=pltpu.CompilerParams(use_tc_tiling_on_sc=False),
        out_type=jax.ShapeDtypeStruct((batch, 128), jnp.float32),
        scratch_types=[
            pltpu.VMEM((per_w, _CH), jnp.int32),
            pltpu.VMEM((per_w * _CH, HID), jnp.float32),
            pltpu.SemaphoreType.DMA,
        ],
    )
    def k(table_hbm, idx_hbm, out_hbm, idx_v, rows_v, sem):
        wid = lax.axis_index("s") * _NC + lax.axis_index("c")
        cbase = wid * per_w
        is_a = cbase < half_c
        lane0 = jnp.where(is_a, 0, _BLANE)
        rsub = jnp.where(is_a, 0, half_c * _CH)
        pltpu.sync_copy(idx_hbm.at[pl.ds(cbase, per_w)], idx_v)
        copies = [
            pltpu.async_copy(table_hbm.at[idx_v.at[j]],
                             rows_v.at[pl.ds(j * _CH, _CH)], sem)
            for j in range(per_w)
        ]
        for c in copies:
            c.wait()
        pltpu.sync_copy(
            rows_v,
            out_hbm.at[pl.ds(cbase * _CH - rsub, per_w * _CH),
                       pl.ds(lane0, HID)])

    return k(table, idx2)


def _mlp_body(g_ref, b1_ref, w2_ref, out_ref):
    g = g_ref[...]
    t = jnp.tanh(g + pltpu.roll(g, 128 - _BLANE, 1) + b1_ref[...])
    lane = lax.broadcasted_iota(jnp.int32, (_BB, 128), 1)
    t = jnp.where(lane < HID, t, jnp.where(lane == HID, 1.0, 0.0))
    out_ref[...] = lax.dot_general(w2_ref[...], t, (((0,), (1,)), ((), ())),
                                   preferred_element_type=jnp.float32)


def _mlp(g2, fc1_b, fc2_w, fc2_b, batch):
    nb = batch // _BB
    b1p = jnp.pad(fc1_b, (0, 128 - HID)).reshape(1, 128)
    w2p = jnp.zeros((128, VOCAB), jnp.float32)
    w2p = w2p.at[:HID].set(fc2_w.T).at[HID].set(fc2_b)
    out_t = pl.pallas_call(
        _mlp_body,
        grid=(nb,),
        in_specs=[
            pl.BlockSpec((_BB, 128), lambda i: (i, 0)),
            pl.BlockSpec((1, 128), lambda i: (0, 0)),
            pl.BlockSpec((128, VOCAB), lambda i: (0, 0)),
        ],
        out_specs=pl.BlockSpec((VOCAB, _BB), lambda i: (0, i)),
        out_shape=jax.ShapeDtypeStruct((VOCAB, batch), jnp.float32),
    )(g2, b1p, w2p)
    return out_t.T


def kernel(x, emb, fc1_w, fc1_b, fc2_w, fc2_b):
    batch = x.shape[0]
    table = _build_tables(emb, fc1_w)
    xi = x.astype(jnp.int32)
    idx = jnp.concatenate([xi[:, 0], xi[:, 1] + VOCAB]).reshape(-1, _CH)
    g = _sc_gather(table, idx, batch)
    return _mlp(g, fc1_b, fc2_w, fc2_b, batch)
